# Initial kernel scaffold; baseline (speedup 1.0000x reference)
import jax
import jax.numpy as jnp
from jax.experimental import pallas as pl


def kernel(x, hub_scores, batch, W1, b1, W2, b2):
    raise NotImplementedError("write your pallas kernel here")


# speedup vs baseline: 4.8650x; 4.8650x over previous
import functools

import jax
import jax.numpy as jnp
from jax.experimental import pallas as pl

N = 100000
D = 512
H = 256
G = 64
BLK = 2000


def _fused_kernel(x_ref, hub_ref, batch_ref, w1_ref, b1_ref, w2_ref, b2_ref,
                  out_ref):
    i = pl.program_id(0)
    x_blk = x_ref[...]
    h = jnp.dot(x_blk, w1_ref[...], preferred_element_type=jnp.float32)
    h = jnp.maximum(h + b1_ref[...][None, :], 0.0)
    logits = jnp.dot(h, w2_ref[...], preferred_element_type=jnp.float32)
    logits = logits + b2_ref[0]
    w = jax.nn.sigmoid(logits) * (1.0 + 2.0 * hub_ref[...])
    gids = jax.lax.broadcasted_iota(jnp.int32, (BLK, G), 1)
    a = jnp.where(batch_ref[...] == gids, w, 0.0)
    partial = jax.lax.dot_general(
        a, x_blk, dimension_numbers=(((0,), (0,)), ((), ())),
        preferred_element_type=jnp.float32)

    @pl.when(i == 0)
    def _init():
        out_ref[...] = partial

    @pl.when(i != 0)
    def _acc():
        out_ref[...] += partial


@jax.jit
def kernel(x, hub_scores, batch, W1, b1, W2, b2):
    hub2 = hub_scores.reshape(N, 1)
    batch2 = batch.astype(jnp.int32).reshape(N, 1)
    grid = N // BLK
    out = pl.pallas_call(
        _fused_kernel,
        grid=(grid,),
        in_specs=[
            pl.BlockSpec((BLK, D), lambda i: (i, 0)),
            pl.BlockSpec((BLK, 1), lambda i: (i, 0)),
            pl.BlockSpec((BLK, 1), lambda i: (i, 0)),
            pl.BlockSpec((D, H), lambda i: (0, 0)),
            pl.BlockSpec((H,), lambda i: (0,)),
            pl.BlockSpec((H, 1), lambda i: (0, 0)),
            pl.BlockSpec((1,), lambda i: (0,)),
        ],
        out_specs=pl.BlockSpec((G, D), lambda i: (0, 0)),
        out_shape=jax.ShapeDtypeStruct((G, D), jnp.float32),
    )(x, hub2, batch2, W1, b1, W2, b2)
    return out
